# baseline (speedup 1.0000x reference)
import functools
import math

import jax
import jax.numpy as jnp
from jax.experimental import pallas as pl
from jax.experimental.pallas import tpu as pltpu

_T = math.e
_AR = 0.7


def _body(x_ref, w_ref, b_ref, o_ref, *, na):
    l = jax.lax.dot_general(
        x_ref[...], w_ref[...],
        (((1,), (1,)), ((), ())),
        preferred_element_type=jnp.float32,
        precision=jax.lax.Precision.DEFAULT,
    )
    scaled = (l + b_ref[...]) * (1.0 / _T)
    m = jnp.max(scaled, axis=1, keepdims=True)
    e = jnp.exp(scaled - m)

    n = e.shape[1]
    ones = jnp.ones((n, 1), jnp.float32)

    def rowsum(v):
        return jax.lax.dot_general(
            v, ones, (((1,), (0,)), ((), ())),
            preferred_element_type=jnp.float32)

    esum = rowsum(e)

    key = jax.lax.bitcast_convert_type(e, jnp.int32)
    fna = jnp.float32(na)

    def step(i, prefix):
        trial = prefix | (jnp.int32(1) << (29 - i))
        cnt = rowsum(jnp.where(key >= trial, 1.0, 0.0))
        return jnp.where(cnt >= fna, trial, prefix)

    thr = jax.lax.fori_loop(
        0, 30, step, jnp.zeros((e.shape[0], 1), jnp.int32))

    kept = key >= thr
    s = rowsum(jnp.where(kept, e, 0.0))
    recip = 1.0 / (s + 1e-8 * esum)
    o_ref[...] = jnp.where(kept, e * recip, 0.0)


def kernel(x, W, b):
    rows, d = x.shape
    n = W.shape[0]
    na = max(1, int(n * _AR))
    bm = 256
    while rows % bm:
        bm //= 2
    grid = (rows // bm,)
    b2 = b.reshape(1, n)
    return pl.pallas_call(
        functools.partial(_body, na=na),
        grid=grid,
        in_specs=[
            pl.BlockSpec((bm, d), lambda i: (i, 0)),
            pl.BlockSpec((n, d), lambda i: (0, 0)),
            pl.BlockSpec((1, n), lambda i: (0, 0)),
        ],
        out_specs=pl.BlockSpec((bm, n), lambda i: (i, 0)),
        out_shape=jax.ShapeDtypeStruct((rows, n), jnp.float32),
        compiler_params=pltpu.CompilerParams(
            dimension_semantics=("parallel",),
        ),
    )(x, W, b2)


# speedup vs baseline: 1.5554x; 1.5554x over previous
import functools
import math

import jax
import jax.numpy as jnp
from jax.experimental import pallas as pl
from jax.experimental.pallas import tpu as pltpu

_T = math.e
_AR = 0.7


def _body(x_ref, w_ref, b_ref, o_ref, *, na):
    l = jax.lax.dot_general(
        x_ref[...], w_ref[...],
        (((1,), (1,)), ((), ())),
        preferred_element_type=jnp.float32,
        precision=jax.lax.Precision.DEFAULT,
    )
    scaled = (l + b_ref[...]) * (1.0 / _T)
    m = jnp.max(scaled, axis=1, keepdims=True)
    e = jnp.exp(scaled - m)

    esum = jnp.sum(e, axis=1, keepdims=True)

    key = jax.lax.bitcast_convert_type(e, jnp.int32)
    bmr = e.shape[0]
    fna = jnp.float32(na)

    def cond(st):
        _, lo, hi, c_lo, _ = st
        return jnp.any((c_lo != fna) & (hi - lo > 1))

    def body(st):
        it, lo, hi, c_lo, c_hi = st
        frac = (fna - c_hi) / jnp.maximum(c_lo - c_hi, 1.0)
        width = hi - lo
        t_key = hi - (width.astype(jnp.float32) * frac).astype(jnp.int32)
        vlo = jax.lax.bitcast_convert_type(lo, jnp.float32)
        vhi = jax.lax.bitcast_convert_type(hi, jnp.float32)
        t_flt = jax.lax.bitcast_convert_type(vhi - (vhi - vlo) * frac,
                                             jnp.int32)
        t = jnp.where(width <= (1 << 16), t_key, t_flt)
        t = jnp.where(it % 6 == 5, lo + (width >> 1), t)
        t = jnp.clip(t, lo + 1, hi - 1)
        cnt = jnp.sum((key >= t).astype(jnp.int32), axis=1,
                      keepdims=True).astype(jnp.float32)
        ge = cnt >= fna
        return (it + 1,
                jnp.where(ge, t, lo), jnp.where(ge, hi, t),
                jnp.where(ge, cnt, c_lo), jnp.where(ge, c_hi, cnt))

    init = (jnp.int32(0),
            jnp.zeros((bmr, 1), jnp.int32),
            jnp.full((bmr, 1), 0x3F800001, jnp.int32),
            jnp.full((bmr, 1), float(e.shape[1]), jnp.float32),
            jnp.zeros((bmr, 1), jnp.float32))
    _, thr, _, _, _ = jax.lax.while_loop(cond, body, init)

    kept = key >= thr
    s = jnp.sum(jnp.where(kept, e, 0.0), axis=1, keepdims=True)
    recip = 1.0 / (s + 1e-8 * esum)
    o_ref[...] = jnp.where(kept, e * recip, 0.0)


def kernel(x, W, b):
    rows, d = x.shape
    n = W.shape[0]
    na = max(1, int(n * _AR))
    bm = 256
    while rows % bm:
        bm //= 2
    grid = (rows // bm,)
    b2 = b.reshape(1, n)
    return pl.pallas_call(
        functools.partial(_body, na=na),
        grid=grid,
        in_specs=[
            pl.BlockSpec((bm, d), lambda i: (i, 0)),
            pl.BlockSpec((n, d), lambda i: (0, 0)),
            pl.BlockSpec((1, n), lambda i: (0, 0)),
        ],
        out_specs=pl.BlockSpec((bm, n), lambda i: (i, 0)),
        out_shape=jax.ShapeDtypeStruct((rows, n), jnp.float32),
        compiler_params=pltpu.CompilerParams(
            dimension_semantics=("parallel",),
        ),
    )(x, W, b2)
